# baseline (speedup 1.0000x reference)
import functools

import jax
import jax.numpy as jnp
from jax import lax
from jax.experimental import pallas as pl
from jax.experimental.pallas import tpu as pltpu
from jax.experimental.pallas import tpu_sc as plsc

E = 8
TOP_K = 2
H = 1024
I = 4096

B = 256
MAXB = 2 * 2048 // B + E
NTOT = MAXB * B
NI = 4
TI = I // NI
CCH = 1024


def _routing_body(x_ref, wg_ref, comb_ref):
    x = x_ref[...]
    logits = lax.dot_general(x, wg_ref[...], (((1,), (1,)), ((), ())),
                             preferred_element_type=jnp.float32)
    logits = 30.0 * jnp.tanh(logits / 30.0)
    m = jnp.max(logits, axis=1, keepdims=True)
    e = jnp.exp(logits - m)
    probs = e / jnp.sum(e, axis=1, keepdims=True)
    iot = lax.broadcasted_iota(jnp.int32, probs.shape, 1)
    m1 = jnp.max(probs, axis=1, keepdims=True)
    i1 = jnp.min(jnp.where(probs == m1, iot, E), axis=1, keepdims=True)
    masked = jnp.where(iot == i1, -jnp.inf, probs)
    m2 = jnp.max(masked, axis=1, keepdims=True)
    i2 = jnp.min(jnp.where(masked == m2, iot, E), axis=1, keepdims=True)
    s = m1 + m2
    comb_ref[...] = jnp.where(iot == i1, m1 / s,
                              jnp.where(iot == i2, m2 / s, 0.0))


def _make_sc_gather(T):
    info = plsc.get_sparse_core_info()
    NC, NS = info.num_cores, info.num_subcores
    NW = NC * NS
    assert NTOT % (8 * NW) == 0
    b_per_w = NTOT // NW
    mesh = plsc.VectorSubcoreMesh(core_axis_name="c", subcore_axis_name="s")

    CH = 32
    assert b_per_w % CH == 0

    @functools.partial(
        pl.kernel, mesh=mesh,
        out_type=jax.ShapeDtypeStruct((NTOT, H), jnp.float32),
        scratch_types=[
            pltpu.VMEM((b_per_w,), jnp.int32),
            pltpu.VMEM((CH, H), jnp.float32),
            pltpu.SemaphoreType.DMA,
        ],
    )
    def gather_k(x_hbm, idx_hbm, out_hbm, idx_v, rows_v, sem):
        wid = lax.axis_index("s") * NC + lax.axis_index("c")
        base = wid * b_per_w
        pltpu.sync_copy(idx_hbm.at[pl.ds(base, b_per_w)], idx_v)
        for c in range(b_per_w // CH):
            pltpu.async_copy(x_hbm.at[idx_v.at[pl.ds(c * CH, CH)]],
                             rows_v, sem).wait()
            pltpu.sync_copy(rows_v, out_hbm.at[pl.ds(base + c * CH, CH)])

    return gather_k


def _moe_body(be_ref, blen_ref, xs_ref, w1_ref, w3_ref, w2_ref,
              rowsf_ref, wf_ref, out_ref, ys_ref):
    i = pl.program_id(0)
    j = pl.program_id(1)

    @pl.when(jnp.logical_and(i == 0, j == 0))
    def _():
        ys_ref[...] = jnp.zeros_like(ys_ref)

    @pl.when(blen_ref[j] > 0)
    def _():
        xb = xs_ref[...].astype(jnp.bfloat16)
        h1 = lax.dot_general(xb, w1_ref[0].astype(jnp.bfloat16),
                             (((1,), (1,)), ((), ())),
                             preferred_element_type=jnp.float32)
        u = lax.dot_general(xb, w3_ref[0].astype(jnp.bfloat16),
                            (((1,), (1,)), ((), ())),
                            preferred_element_type=jnp.float32)
        g = h1 * 0.5 * (1.0 + lax.erf(h1 * 0.7071067811865476))
        act = (g * u).astype(jnp.bfloat16)
        contrib = lax.dot_general(act, w2_ref[0].astype(jnp.bfloat16),
                                  (((1,), (1,)), ((), ())),
                                  preferred_element_type=jnp.float32)
        ys_ref[pl.ds(j * B, B), :] += contrib.astype(jnp.bfloat16)

    @pl.when(jnp.logical_and(i == NI - 1, j == MAXB - 1))
    def _():
        T = out_ref.shape[0]
        out_ref[...] = jnp.zeros_like(out_ref)
        for c in range(NTOT // CCH):
            rows_c = rowsf_ref[0, pl.ds(c * CCH, CCH)]
            w_c = wf_ref[0, pl.ds(c * CCH, CCH)]
            tio = lax.broadcasted_iota(jnp.int32, (T, CCH), 0)
            ptw = jnp.where(tio == rows_c[None, :], w_c[None, :],
                            0.0).astype(jnp.bfloat16)
            ysc = ys_ref[pl.ds(c * CCH, CCH), :]
            out_ref[...] += lax.dot_general(ptw, ysc, (((1,), (0,)), ((), ())),
                                            preferred_element_type=jnp.float32)


def kernel(hidden_states, Wg, W1, W3, W2):
    orig_shape = hidden_states.shape
    x = hidden_states.reshape(-1, H)
    T = x.shape[0]

    comb = pl.pallas_call(
        _routing_body,
        out_shape=jax.ShapeDtypeStruct((T, E), jnp.float32),
    )(x, Wg)

    i32 = jnp.int32
    mask = comb > 0.0
    counts = jnp.sum(mask.astype(i32), axis=0)
    nblk = (counts + B - 1) // B
    cumblk = jnp.cumsum(nblk).astype(i32)
    blkbase = jnp.concatenate([jnp.zeros((1,), i32), cumblk[:-1]])
    within = jnp.cumsum(mask.astype(i32), axis=0) - 1
    ppos = blkbase[None, :] * B + within
    posf = jnp.where(mask, ppos, NTOT).reshape(-1)
    tokf = jnp.broadcast_to(jnp.arange(T, dtype=i32)[:, None], (T, E)).reshape(-1)
    rows_flat = jnp.zeros((NTOT,), i32).at[posf].set(tokf, mode="drop")
    w_flat = jnp.zeros((NTOT,), jnp.float32).at[posf].set(
        comb.reshape(-1), mode="drop")

    jidx = jnp.arange(MAXB, dtype=i32)
    bexp = jnp.searchsorted(cumblk, jidx, side="right").astype(i32)
    bec = jnp.minimum(bexp, E - 1)
    prev = jnp.where(bec > 0, cumblk[jnp.maximum(bec - 1, 0)], 0)
    kk = jidx - prev
    blen = jnp.clip(counts[bec] - kk * B, 0, B).astype(i32)

    xs = _make_sc_gather(T)(x, rows_flat)

    grid_spec = pltpu.PrefetchScalarGridSpec(
        num_scalar_prefetch=2,
        grid=(NI, MAXB),
        in_specs=[
            pl.BlockSpec((B, H), lambda i, j, be, bl: (j, 0)),
            pl.BlockSpec((1, TI, H), lambda i, j, be, bl: (be[j], i, 0)),
            pl.BlockSpec((1, TI, H), lambda i, j, be, bl: (be[j], i, 0)),
            pl.BlockSpec((1, H, TI), lambda i, j, be, bl: (be[j], 0, i)),
            pl.BlockSpec((1, NTOT), lambda i, j, be, bl: (0, 0)),
            pl.BlockSpec((1, NTOT), lambda i, j, be, bl: (0, 0)),
        ],
        out_specs=pl.BlockSpec((T, H), lambda i, j, be, bl: (0, 0)),
        scratch_shapes=[
            pltpu.VMEM((NTOT, H), jnp.bfloat16),
        ],
    )
    out = pl.pallas_call(
        _moe_body,
        grid_spec=grid_spec,
        out_shape=jax.ShapeDtypeStruct((T, H), jnp.float32),
    )(bec, blen, xs, W1, W3, W2,
      rows_flat.reshape(1, NTOT), w_flat.reshape(1, NTOT))

    return out.reshape(orig_shape)
